# baseline (speedup 1.0000x reference)
import functools

import jax
import jax.numpy as jnp
from jax import lax
from jax.experimental import pallas as pl
from jax.experimental.pallas import tpu as pltpu
from jax.experimental.pallas import tpu_sc as plsc

_D_MIN = 0.001
_LAMB = 0.85
_ALPHA = 10.0
_BETA1 = 0.1
_BETA2 = 0.001
_SENTINEL = 1e9
_BIG = 1e10

_P = 50176
_T = 3584
_K = 128
_U = 8

_NBLK_TC = 12
_PTC = _NBLK_TC * _T
_PSC = _P - _PTC
_NW = 32
_PW = _PSC * _U // _NW
_NPV = _PW // 16
_G = 16


def _tc_body(pred_ref, targ_ref, lid_ref, tflat_ref, lflat_ref, bm_ref,
             scal_ref, outx_ref, a_ref):
    p = pred_ref[...]
    t = targ_ref[...]
    mask = jnp.logical_and(p >= _D_MIN, t >= _D_MIN)
    g = jnp.where(mask, jnp.log(p + 1e-5) - jnp.log(t + 1e-5), 0.0)
    n = p.size
    sum_g = jnp.sum(g)
    sum_g2 = jnp.sum(g * g)
    mean_g = sum_g / n
    var_g = (sum_g2 - n * mean_g * mean_g) / (n - 1)
    dg = var_g + (1.0 - _LAMB) * mean_g * mean_g
    scal_ref[0, 0] = jnp.sqrt(dg)

    lv = lid_ref[...]
    for u in range(4):
        scal_ref[0, 16 + u] = jnp.sum((t[u] >= _D_MIN).astype(jnp.float32))
        scal_ref[0, 20 + u] = jnp.sum((lv[u] >= _D_MIN).astype(jnp.float32))

    xt = tflat_ref[...]
    xl = lflat_ref[...]
    xst = jnp.where(xt >= _D_MIN, xt, _SENTINEL)
    xsl = jnp.where(xl >= _D_MIN, xl, _SENTINEL)
    a_ref[0:4, :] = xst.astype(jnp.bfloat16)
    a_ref[4:8, :] = xsl.astype(jnp.bfloat16)
    a_ref[8:12, :] = (xst * xst).astype(jnp.bfloat16)
    a_ref[12:16, :] = (xsl * xsl).astype(jnp.bfloat16)
    a_ref[16:17, :] = jnp.ones((1, _P), jnp.bfloat16)
    a_ref[17:24, :] = jnp.zeros((7, _P), jnp.bfloat16)

    def blk_body(jj, carry):
        minxs, sys_ = carry
        minxs = list(minxs)
        sys_ = list(sys_)
        bmv = bm_ref[...]
        for b in range(2):
            j = jj * 2 + b
            ablk = a_ref[:, pl.ds(j * _T, _T)]
            for h in range(2):
                d_h = jax.lax.dot_general(
                    bmv[h * 512:(h + 1) * 512, :], ablk,
                    (((1,), (0,)), ((), ())),
                    preferred_element_type=jnp.float32)
                for u4 in range(4):
                    u = h * 4 + u4
                    du = d_h[u4 * _K:(u4 + 1) * _K, :]
                    mx = minxs[u]
                    my_tiles = []
                    for i in range(_T // 128):
                        col = du[:, i * 128:(i + 1) * 128]
                        mx = jnp.minimum(mx, col)
                        f = col[0:8, :]
                        for r in range(1, _K // 8):
                            f = jnp.minimum(f, col[r * 8:(r + 1) * 8, :])
                        f = jnp.minimum(f, 2.0)
                        my_tiles.append(
                            jnp.min(f, axis=0, keepdims=True))
                    minxs[u] = mx
                    acc = my_tiles[0]
                    for mt in my_tiles[1:]:
                        acc = acc + mt
                    sys_[u] = sys_[u] + jnp.sum(acc)
        return tuple(minxs), tuple(sys_)

    minx0 = tuple(
        jnp.full((_K, 128), jnp.float32(3e38)) for _ in range(_U))
    sy0 = tuple(jnp.float32(0.0) for _ in range(_U))
    minxs, sys_ = jax.lax.fori_loop(0, _NBLK_TC // 2, blk_body, (minx0, sy0))

    for u in range(_U):
        outx_ref[u:u + 1, :] = jnp.min(minxs[u], axis=1).reshape(1, _K)
        scal_ref[0, 1 + u] = sys_[u]


def _sc_body(pts_ref, cb_ref, outx_ref, outy_ref, xv, cbv, myv, oxv, oyv):
    cid = lax.axis_index("c")
    sid = lax.axis_index("s")
    wid = sid * 2 + cid
    u = wid // 4
    b = lax.rem(u, 4)
    base = _PTC + lax.rem(wid, 4) * _PW
    pltpu.sync_copy(pts_ref.at[u, pl.ds(base, _PW)], xv)
    pltpu.sync_copy(cb_ref.at[b], cbv)

    big = jnp.full((16,), 3e38, jnp.float32)

    def prep(i, carry):
        x = xv[pl.ds(i * 16, 16)]
        xv[pl.ds(i * 16, 16)] = jnp.where(x >= _D_MIN, x, _SENTINEL)
        myv[pl.ds(i * 16, 16)] = big
        return carry

    lax.fori_loop(0, _NPV, prep, jnp.float32(0.0))

    for grp in range(_K // _G):
        cbs = [cbv[grp * _G + k] for k in range(_G)]

        def pv_body(i, accs):
            x = xv[pl.ds(i * 16, 16)]
            my = myv[pl.ds(i * 16, 16)]
            new_accs = []
            for k in range(_G):
                d = x - cbs[k]
                d = d * d
                my = jnp.minimum(my, d)
                new_accs.append(jnp.minimum(accs[k], d))
            myv[pl.ds(i * 16, 16)] = my
            return tuple(new_accs)

        accs = lax.fori_loop(0, _NPV, pv_body, tuple([big] * _G))
        for k in range(_G):
            oxv[grp * _G + k] = accs[k]

    def sum_body(i, sacc):
        return sacc + jnp.minimum(myv[pl.ds(i * 16, 16)], 2.0)

    sacc = lax.fori_loop(0, _NPV, sum_body, jnp.zeros((16,), jnp.float32))
    oyv[...] = sacc
    pltpu.sync_copy(oxv, outx_ref.at[wid])
    pltpu.sync_copy(oyv, outy_ref.at[wid])


@functools.partial(jax.jit, static_argnames=())
def kernel(predict, centers, target, lidar):
    B = predict.shape[0]
    tflat = target.reshape(B, _P)
    lflat = lidar.reshape(B, _P)
    pts8 = jnp.concatenate([tflat, lflat], axis=0)
    cent_all = jnp.concatenate([centers, centers], axis=0)
    eye = jnp.eye(_U, dtype=jnp.float32)
    bm = jnp.concatenate(
        [
            (-2.0 * cent_all)[:, :, None] * eye[:, None, :],
            jnp.broadcast_to(eye[:, None, :], (_U, _K, _U)),
            (cent_all * cent_all)[:, :, None],
            jnp.zeros((_U, _K, 7), jnp.float32),
        ],
        axis=2,
    ).astype(jnp.bfloat16).reshape(_U * _K, 24)
    cbig = jnp.broadcast_to(centers[:, :, None], (B, _K, 16))

    sc_call = pl.kernel(
        _sc_body,
        out_type=(
            jax.ShapeDtypeStruct((_NW, _K, 16), jnp.float32),
            jax.ShapeDtypeStruct((_NW, 16), jnp.float32),
        ),
        mesh=plsc.VectorSubcoreMesh(core_axis_name="c", subcore_axis_name="s"),
        scratch_types=[
            pltpu.VMEM((_PW,), jnp.float32),
            pltpu.VMEM((_K, 16), jnp.float32),
            pltpu.VMEM((_PW,), jnp.float32),
            pltpu.VMEM((_K, 16), jnp.float32),
            pltpu.VMEM((16,), jnp.float32),
        ],
    )
    scx, scy = sc_call(pts8, cbig)

    scal, tcx = pl.pallas_call(
        _tc_body,
        out_shape=(
            jax.ShapeDtypeStruct((1, 32), jnp.float32),
            jax.ShapeDtypeStruct((_U, _K), jnp.float32),
        ),
        in_specs=[pl.BlockSpec(memory_space=pltpu.VMEM)] * 6,
        out_specs=(
            pl.BlockSpec(memory_space=pltpu.SMEM),
            pl.BlockSpec(memory_space=pltpu.VMEM),
        ),
        scratch_shapes=[pltpu.VMEM((24, _P), jnp.bfloat16)],
    )(predict, target, lidar, tflat, lflat, bm)

    sil = scal[0, 0]
    sy_tc = scal[0, 1:9]
    counts = scal[0, 16:24]
    minx = jnp.minimum(
        tcx, jnp.min(scx.reshape(_U, 4, _K, 16), axis=(1, 3)))
    cham_x = jnp.mean(jnp.minimum(minx, _BIG), axis=1)
    sy = sy_tc + jnp.sum(scy.reshape(_U, 4 * 16), axis=1)
    cham_y = (sy - 2.0 * (_P - counts)) / jnp.maximum(counts, 1.0)
    w = jnp.array([_BETA1] * 4 + [_BETA2] * 4, jnp.float32) * 0.25
    return _ALPHA * sil + jnp.sum(w * (cham_x + cham_y))


# speedup vs baseline: 1.0000x; 1.0000x over previous
import functools

import jax
import jax.numpy as jnp
from jax import lax
from jax.experimental import pallas as pl
from jax.experimental.pallas import tpu as pltpu
from jax.experimental.pallas import tpu_sc as plsc

_D_MIN = 0.001
_LAMB = 0.85
_ALPHA = 10.0
_BETA1 = 0.1
_BETA2 = 0.001
_SENTINEL = 1e9
_BIG = 1e10

_P = 50176
_T = 3584
_K = 128
_U = 8

_NBLK_TC = 12
_PTC = _NBLK_TC * _T
_PSC = _P - _PTC
_NW = 32
_PW = _PSC * _U // _NW
_NPV = _PW // 16
_G = 16


def _tc_body(pred_ref, targ_ref, lid_ref, tflat_ref, lflat_ref, bm_ref,
             scal_ref, outx_ref, a_ref):
    p = pred_ref[...]
    t = targ_ref[...]
    mask = jnp.logical_and(p >= _D_MIN, t >= _D_MIN)
    g = jnp.where(mask, jnp.log(p + 1e-5) - jnp.log(t + 1e-5), 0.0)
    n = p.size
    sum_g = jnp.sum(g)
    sum_g2 = jnp.sum(g * g)
    mean_g = sum_g / n
    var_g = (sum_g2 - n * mean_g * mean_g) / (n - 1)
    dg = var_g + (1.0 - _LAMB) * mean_g * mean_g
    scal_ref[0, 0] = jnp.sqrt(dg)

    lv = lid_ref[...]
    for u in range(4):
        scal_ref[0, 16 + u] = jnp.sum((t[u] >= _D_MIN).astype(jnp.float32))
        scal_ref[0, 20 + u] = jnp.sum((lv[u] >= _D_MIN).astype(jnp.float32))

    xt = tflat_ref[...]
    xl = lflat_ref[...]
    xst = jnp.where(xt >= _D_MIN, xt, _SENTINEL)
    xsl = jnp.where(xl >= _D_MIN, xl, _SENTINEL)
    a_ref[0:4, :] = xst.astype(jnp.bfloat16)
    a_ref[4:8, :] = xsl.astype(jnp.bfloat16)
    a_ref[8:12, :] = (xst * xst).astype(jnp.bfloat16)
    a_ref[12:16, :] = (xsl * xsl).astype(jnp.bfloat16)
    a_ref[16:17, :] = jnp.ones((1, _P), jnp.bfloat16)
    a_ref[17:24, :] = jnp.zeros((7, _P), jnp.bfloat16)

    def blk_body(jj, carry):
        minxs, sys_ = carry
        minxs = list(minxs)
        sys_ = list(sys_)
        bmv = bm_ref[...]
        for b in range(2):
            j = jj * 2 + b
            ablk = a_ref[:, pl.ds(j * _T, _T)]
            for h in range(2):
                d_h = jax.lax.dot_general(
                    bmv[h * 512:(h + 1) * 512, :], ablk,
                    (((1,), (0,)), ((), ())),
                    preferred_element_type=jnp.float32)
                for u4 in range(4):
                    u = h * 4 + u4
                    du = d_h[u4 * _K:(u4 + 1) * _K, :]
                    mx = minxs[u]
                    my_tiles = []
                    for i in range(_T // 128):
                        col = du[:, i * 128:(i + 1) * 128]
                        mx = jnp.minimum(mx, col)
                        f = col[0:8, :]
                        for r in range(1, _K // 8):
                            f = jnp.minimum(f, col[r * 8:(r + 1) * 8, :])
                        f = jnp.minimum(f, 2.0)
                        my_tiles.append(
                            jnp.min(f, axis=0, keepdims=True))
                    minxs[u] = mx
                    acc = my_tiles[0]
                    for mt in my_tiles[1:]:
                        acc = acc + mt
                    sys_[u] = sys_[u] + jnp.sum(acc)
        return tuple(minxs), tuple(sys_)

    minx0 = tuple(
        jnp.full((_K, 128), jnp.float32(3e38)) for _ in range(_U))
    sy0 = tuple(jnp.float32(0.0) for _ in range(_U))
    minxs, sys_ = jax.lax.fori_loop(0, _NBLK_TC // 2, blk_body, (minx0, sy0))

    for u in range(_U):
        outx_ref[u:u + 1, :] = jnp.min(minxs[u], axis=1).reshape(1, _K)
        scal_ref[0, 1 + u] = sys_[u]


def _sc_body(pts_ref, cb_ref, outx_ref, outy_ref, xv, cbv, myv, oxv, oyv):
    cid = lax.axis_index("c")
    sid = lax.axis_index("s")
    wid = sid * 2 + cid
    u = wid // 4
    b = lax.rem(u, 4)
    base = _PTC + lax.rem(wid, 4) * _PW
    pltpu.sync_copy(pts_ref.at[u, pl.ds(base, _PW)], xv)
    pltpu.sync_copy(cb_ref.at[b], cbv)

    big = jnp.full((16,), 3e38, jnp.float32)

    def prep(i, carry):
        x = xv[pl.ds(i * 16, 16)]
        xv[pl.ds(i * 16, 16)] = jnp.where(x >= _D_MIN, x, _SENTINEL)
        myv[pl.ds(i * 16, 16)] = big
        return carry

    lax.fori_loop(0, _NPV, prep, jnp.float32(0.0))

    for grp in range(_K // _G):
        cbs = [cbv[grp * _G + k] for k in range(_G)]

        def pv_body(i, accs):
            x = xv[pl.ds(i * 16, 16)]
            my = myv[pl.ds(i * 16, 16)]
            new_accs = []
            for k in range(_G):
                d = x - cbs[k]
                d = d * d
                my = jnp.minimum(my, d)
                new_accs.append(jnp.minimum(accs[k], d))
            myv[pl.ds(i * 16, 16)] = my
            return tuple(new_accs)

        accs = lax.fori_loop(0, _NPV, pv_body, tuple([big] * _G))
        for k in range(_G):
            oxv[grp * _G + k] = accs[k]

    def sum_body(i, sacc):
        return sacc + jnp.minimum(myv[pl.ds(i * 16, 16)], 2.0)

    sacc = lax.fori_loop(0, _NPV, sum_body, jnp.zeros((16,), jnp.float32))
    oyv[...] = sacc
    pltpu.sync_copy(oxv, outx_ref.at[wid])
    pltpu.sync_copy(oyv, outy_ref.at[wid])


@functools.partial(jax.jit, static_argnames=())
def kernel(predict, centers, target, lidar):
    B = predict.shape[0]
    tflat = target.reshape(B, _P)
    lflat = lidar.reshape(B, _P)
    pts8 = jnp.concatenate([tflat, lflat], axis=0)
    cent_all = jnp.concatenate([centers, centers], axis=0)
    eye = jnp.eye(_U, dtype=jnp.float32)
    bm = jnp.concatenate(
        [
            (-2.0 * cent_all)[:, :, None] * eye[:, None, :],
            jnp.broadcast_to(eye[:, None, :], (_U, _K, _U)),
            (cent_all * cent_all)[:, :, None],
            jnp.zeros((_U, _K, 7), jnp.float32),
        ],
        axis=2,
    ).astype(jnp.bfloat16).reshape(_U * _K, 24)
    cbig = jnp.broadcast_to(centers[:, :, None], (B, _K, 16))

    sc_call = pl.kernel(
        _sc_body,
        out_type=(
            jax.ShapeDtypeStruct((_NW, _K, 16), jnp.float32),
            jax.ShapeDtypeStruct((_NW, 16), jnp.float32),
        ),
        mesh=plsc.VectorSubcoreMesh(core_axis_name="c", subcore_axis_name="s"),
        scratch_types=[
            pltpu.VMEM((_PW,), jnp.float32),
            pltpu.VMEM((_K, 16), jnp.float32),
            pltpu.VMEM((_PW,), jnp.float32),
            pltpu.VMEM((_K, 16), jnp.float32),
            pltpu.VMEM((16,), jnp.float32),
        ],
    )
    scal, tcx = pl.pallas_call(
        _tc_body,
        out_shape=(
            jax.ShapeDtypeStruct((1, 32), jnp.float32),
            jax.ShapeDtypeStruct((_U, _K), jnp.float32),
        ),
        in_specs=[pl.BlockSpec(memory_space=pltpu.VMEM)] * 6,
        out_specs=(
            pl.BlockSpec(memory_space=pltpu.SMEM),
            pl.BlockSpec(memory_space=pltpu.VMEM),
        ),
        scratch_shapes=[pltpu.VMEM((24, _P), jnp.bfloat16)],
    )(predict, target, lidar, tflat, lflat, bm)

    scx, scy = sc_call(pts8, cbig)

    sil = scal[0, 0]
    sy_tc = scal[0, 1:9]
    counts = scal[0, 16:24]
    minx = jnp.minimum(
        tcx, jnp.min(scx.reshape(_U, 4, _K, 16), axis=(1, 3)))
    cham_x = jnp.mean(jnp.minimum(minx, _BIG), axis=1)
    sy = sy_tc + jnp.sum(scy.reshape(_U, 4 * 16), axis=1)
    cham_y = (sy - 2.0 * (_P - counts)) / jnp.maximum(counts, 1.0)
    w = jnp.array([_BETA1] * 4 + [_BETA2] * 4, jnp.float32) * 0.25
    return _ALPHA * sil + jnp.sum(w * (cham_x + cham_y))


# speedup vs baseline: 1.3963x; 1.3963x over previous
import functools

import jax
import jax.numpy as jnp
from jax.experimental import pallas as pl
from jax.experimental.pallas import tpu as pltpu

_D_MIN = 0.001
_LAMB = 0.85
_ALPHA = 10.0
_BETA1 = 0.1
_BETA2 = 0.001
_SENTINEL = 1e9
_BIG = 1e10

_P = 50176
_T = 3584
_NBLK = _P // _T
_K = 128
_U = 8


def _body(pred_ref, targ_ref, lid_ref, tflat_ref, lflat_ref, bm_ref,
          out_ref, at_ref, al_ref):
    p = pred_ref[...]
    t = targ_ref[...]
    mask = jnp.logical_and(p >= _D_MIN, t >= _D_MIN)
    g = jnp.where(mask, jnp.log(p + 1e-5) - jnp.log(t + 1e-5), 0.0)
    n = p.size
    sum_g = jnp.sum(g)
    sum_g2 = jnp.sum(g * g)
    mean_g = sum_g / n
    var_g = (sum_g2 - n * mean_g * mean_g) / (n - 1)
    dg = var_g + (1.0 - _LAMB) * mean_g * mean_g
    sil = jnp.sqrt(dg)

    lv = lid_ref[...]
    counts = [jnp.sum((t[u] >= _D_MIN).astype(jnp.float32))
              for u in range(4)]
    counts += [jnp.sum((lv[u] >= _D_MIN).astype(jnp.float32))
               for u in range(4)]

    xt = tflat_ref[...]
    xl = lflat_ref[...]
    xst = jnp.where(xt >= _D_MIN, xt, _SENTINEL)
    xsl = jnp.where(xl >= _D_MIN, xl, _SENTINEL)
    at_ref[0:4, :] = xst.astype(jnp.bfloat16)
    at_ref[4:8, :] = (xst * xst).astype(jnp.bfloat16)
    at_ref[8:9, :] = jnp.ones((1, _P), jnp.bfloat16)
    al_ref[0:4, :] = xsl.astype(jnp.bfloat16)
    al_ref[4:8, :] = (xsl * xsl).astype(jnp.bfloat16)
    al_ref[8:9, :] = jnp.ones((1, _P), jnp.bfloat16)

    def blk_body(jj, carry):
        minxs, sys_ = carry
        minxs = list(minxs)
        sys_ = list(sys_)
        bmv = bm_ref[...]
        for b in range(2):
            j = jj * 2 + b
            for h, aref in ((0, at_ref), (1, al_ref)):
                ablk = aref[:, pl.ds(j * _T, _T)]
                d_h = jax.lax.dot_general(
                    bmv, ablk, (((1,), (0,)), ((), ())),
                    preferred_element_type=jnp.float32)
                for u4 in range(4):
                    u = h * 4 + u4
                    du = d_h[u4 * _K:(u4 + 1) * _K, :]
                    mx = minxs[u]
                    my_tiles = []
                    for i in range(_T // 128):
                        col = du[:, i * 128:(i + 1) * 128]
                        mx = jnp.minimum(mx, col)
                        f = col[0:8, :]
                        for r in range(1, _K // 8):
                            f = jnp.minimum(f, col[r * 8:(r + 1) * 8, :])
                        f = jnp.minimum(f, 2.0)
                        my_tiles.append(
                            jnp.min(f, axis=0, keepdims=True))
                    minxs[u] = mx
                    acc = my_tiles[0]
                    for mt in my_tiles[1:]:
                        acc = acc + mt
                    sys_[u] = sys_[u] + jnp.sum(acc)
        return tuple(minxs), tuple(sys_)

    minx0 = tuple(
        jnp.full((_K, 128), jnp.float32(3e38)) for _ in range(_U))
    sy0 = tuple(jnp.float32(0.0) for _ in range(_U))
    minxs, sys_ = jax.lax.fori_loop(0, _NBLK // 2, blk_body, (minx0, sy0))

    cham = jnp.float32(0.0)
    for u in range(_U):
        minx_u = jnp.min(minxs[u], axis=1)
        cham_x = jnp.sum(jnp.minimum(minx_u, _BIG)) / _K
        sy_u = sys_[u] - 2.0 * (_P - counts[u])
        cham_y = sy_u / jnp.maximum(counts[u], 1.0)
        w = (_BETA1 if u < 4 else _BETA2) * 0.25
        cham = cham + w * (cham_x + cham_y)

    out_ref[0, 0] = _ALPHA * sil + cham


@functools.partial(jax.jit, static_argnames=())
def kernel(predict, centers, target, lidar):
    B = predict.shape[0]
    tflat = target.reshape(B, _P)
    lflat = lidar.reshape(B, _P)
    eye = jnp.eye(4, dtype=jnp.float32)
    bm = jnp.concatenate(
        [
            (-2.0 * centers)[:, :, None] * eye[:, None, :],
            jnp.broadcast_to(eye[:, None, :], (4, _K, 4)),
            (centers * centers)[:, :, None],
        ],
        axis=2,
    ).astype(jnp.bfloat16).reshape(4 * _K, 9)

    out = pl.pallas_call(
        _body,
        out_shape=jax.ShapeDtypeStruct((1, 1), jnp.float32),
        in_specs=[pl.BlockSpec(memory_space=pltpu.VMEM)] * 6,
        out_specs=pl.BlockSpec(memory_space=pltpu.SMEM),
        scratch_shapes=[
            pltpu.VMEM((9, _P), jnp.bfloat16),
            pltpu.VMEM((9, _P), jnp.bfloat16),
        ],
    )(predict, target, lidar, tflat, lflat, bm)
    return out[0, 0]


# speedup vs baseline: 1.4051x; 1.0063x over previous
import functools

import jax
import jax.numpy as jnp
from jax.experimental import pallas as pl
from jax.experimental.pallas import tpu as pltpu

_D_MIN = 0.001
_LAMB = 0.85
_ALPHA = 10.0
_BETA1 = 0.1
_BETA2 = 0.001
_SENTINEL = 1e9
_BIG = 1e10

_P = 50176
_T = 7168
_NBLK = _P // _T
_K = 128
_U = 8


def _body(pred_ref, targ_ref, lid_ref, tflat_ref, lflat_ref, bm_ref,
          out_ref, at_ref, al_ref):
    p = pred_ref[...]
    t = targ_ref[...]
    mask = jnp.logical_and(p >= _D_MIN, t >= _D_MIN)
    g = jnp.where(mask, jnp.log(p + 1e-5) - jnp.log(t + 1e-5), 0.0)
    n = p.size
    sum_g = jnp.sum(g)
    sum_g2 = jnp.sum(g * g)
    mean_g = sum_g / n
    var_g = (sum_g2 - n * mean_g * mean_g) / (n - 1)
    dg = var_g + (1.0 - _LAMB) * mean_g * mean_g
    sil = jnp.sqrt(dg)

    lv = lid_ref[...]
    counts = [jnp.sum((t[u] >= _D_MIN).astype(jnp.float32))
              for u in range(4)]
    counts += [jnp.sum((lv[u] >= _D_MIN).astype(jnp.float32))
               for u in range(4)]

    xt = tflat_ref[...]
    xl = lflat_ref[...]
    xst = jnp.where(xt >= _D_MIN, xt, _SENTINEL)
    xsl = jnp.where(xl >= _D_MIN, xl, _SENTINEL)
    at_ref[0:4, :] = xst.astype(jnp.bfloat16)
    at_ref[4:8, :] = (xst * xst).astype(jnp.bfloat16)
    at_ref[8:9, :] = jnp.ones((1, _P), jnp.bfloat16)
    al_ref[0:4, :] = xsl.astype(jnp.bfloat16)
    al_ref[4:8, :] = (xsl * xsl).astype(jnp.bfloat16)
    al_ref[8:9, :] = jnp.ones((1, _P), jnp.bfloat16)

    def blk_body(jj, carry):
        minxs, sys_ = carry
        minxs = list(minxs)
        sys_ = list(sys_)
        bmv = bm_ref[...]
        for b in range(1):
            j = jj
            for h, aref in ((0, at_ref), (1, al_ref)):
                ablk = aref[:, pl.ds(j * _T, _T)]
                d_h = jax.lax.dot_general(
                    bmv, ablk, (((1,), (0,)), ((), ())),
                    preferred_element_type=jnp.float32)
                for u4 in range(4):
                    u = h * 4 + u4
                    du = d_h[u4 * _K:(u4 + 1) * _K, :]
                    mx = minxs[u]
                    my_tiles = []
                    for i in range(_T // 128):
                        col = du[:, i * 128:(i + 1) * 128]
                        mx = jnp.minimum(mx, col)
                        f = col[0:8, :]
                        for r in range(1, _K // 8):
                            f = jnp.minimum(f, col[r * 8:(r + 1) * 8, :])
                        f = jnp.minimum(f, 2.0)
                        my_tiles.append(
                            jnp.min(f, axis=0, keepdims=True))
                    minxs[u] = mx
                    acc = my_tiles[0]
                    for mt in my_tiles[1:]:
                        acc = acc + mt
                    sys_[u] = sys_[u] + jnp.sum(acc)
        return tuple(minxs), tuple(sys_)

    minx0 = tuple(
        jnp.full((_K, 128), jnp.float32(3e38)) for _ in range(_U))
    sy0 = tuple(jnp.float32(0.0) for _ in range(_U))
    minxs, sys_ = jax.lax.fori_loop(0, _NBLK, blk_body, (minx0, sy0))

    cham = jnp.float32(0.0)
    for u in range(_U):
        minx_u = jnp.min(minxs[u], axis=1)
        cham_x = jnp.sum(jnp.minimum(minx_u, _BIG)) / _K
        sy_u = sys_[u] - 2.0 * (_P - counts[u])
        cham_y = sy_u / jnp.maximum(counts[u], 1.0)
        w = (_BETA1 if u < 4 else _BETA2) * 0.25
        cham = cham + w * (cham_x + cham_y)

    out_ref[0, 0] = _ALPHA * sil + cham


@functools.partial(jax.jit, static_argnames=())
def kernel(predict, centers, target, lidar):
    B = predict.shape[0]
    tflat = target.reshape(B, _P)
    lflat = lidar.reshape(B, _P)
    eye = jnp.eye(4, dtype=jnp.float32)
    bm = jnp.concatenate(
        [
            (-2.0 * centers)[:, :, None] * eye[:, None, :],
            jnp.broadcast_to(eye[:, None, :], (4, _K, 4)),
            (centers * centers)[:, :, None],
        ],
        axis=2,
    ).astype(jnp.bfloat16).reshape(4 * _K, 9)

    out = pl.pallas_call(
        _body,
        out_shape=jax.ShapeDtypeStruct((1, 1), jnp.float32),
        in_specs=[pl.BlockSpec(memory_space=pltpu.VMEM)] * 6,
        out_specs=pl.BlockSpec(memory_space=pltpu.SMEM),
        scratch_shapes=[
            pltpu.VMEM((9, _P), jnp.bfloat16),
            pltpu.VMEM((9, _P), jnp.bfloat16),
        ],
    )(predict, target, lidar, tflat, lflat, bm)
    return out[0, 0]
